# Initial kernel scaffold; baseline (speedup 1.0000x reference)
import jax
import jax.numpy as jnp
from jax.experimental import pallas as pl


def kernel(x, knots, poly_params, mixture_weights, integrals_2dgrid):
    raise NotImplementedError("write your pallas kernel here")


# speedup vs baseline: 143.7308x; 143.7308x over previous
import functools

import jax
import jax.numpy as jnp
from jax import lax
from jax.experimental import pallas as pl
from jax.experimental.pallas import tpu as pltpu
from jax.experimental.pallas import tpu_sc as plsc

N = 2097152
K = 1024
M = 8
KB = K - 1

NC, NS, L = 2, 16, 16
NW = NC * NS
PW = N // NW
CHUNK = 2048
NCHUNK = PW // CHUNK
GSUB = 128
NG = CHUNK // GSUB

_LN2 = 0.6931471805599453
_SQRT_HALF_HI = 1.4142135


def _lower_bound(kref, xv):
    base = jnp.zeros((L,), jnp.int32)
    length = K
    while length > 1:
        half = length // 2
        probe = plsc.load_gather(kref, [base + (half - 1)])
        base = jnp.where(probe < xv, base + half, base)
        length -= half
    probe = plsc.load_gather(kref, [base])
    return base + (probe < xv).astype(jnp.int32)


def _log_f32(v):
    b = plsc.bitcast(v, jnp.int32)
    e = (b >> 23) - 127
    m = plsc.bitcast((b & 0x007FFFFF) | 0x3F800000, jnp.float32)
    adj = m > _SQRT_HALF_HI
    m = jnp.where(adj, m * 0.5, m)
    ef = (e + adj.astype(jnp.int32)).astype(jnp.float32)
    s = (m - 1.0) / (m + 1.0)
    s2 = s * s
    r = s * (2.0 + s2 * (2.0 / 3.0 + s2 * (2.0 / 5.0 + s2 * (2.0 / 7.0))))
    return ef * _LN2 + r


def _sc_kernel(x0_h, x1_h, k0_h, k1_h, par_h, zt_h, out_h,
               k0v, k1v, parv, xv0, xv1, e0v, e1v, idxv, zv, outv, sem):
    wid = lax.axis_index("s") * NC + lax.axis_index("c")
    base0 = wid * PW

    pltpu.sync_copy(k0_h, k0v)
    pltpu.sync_copy(k1_h, k1v)
    pltpu.sync_copy(par_h, parv)

    iota = lax.iota(jnp.int32, L)
    cv = [[parv[pl.ds((m * 9 + d) * 128, L)] for d in range(9)] for m in range(M)]

    def chunk_body(ci, carry):
        cbase = base0 + ci * CHUNK
        pltpu.sync_copy(x0_h.at[pl.ds(cbase, CHUNK)], xv0)
        pltpu.sync_copy(x1_h.at[pl.ds(cbase, CHUNK)], xv1)

        def pass1(vi, c):
            off = vi * L
            xa = xv0[pl.ds(off, L)]
            xb = xv1[pl.ds(off, L)]
            ia = jnp.clip(_lower_bound(k0v, xa) - 1, 0, KB - 1)
            ib = jnp.clip(_lower_bound(k1v, xb) - 1, 0, KB - 1)
            ea = xa - plsc.load_gather(k0v, [ia])
            eb = xb - plsc.load_gather(k1v, [ib])
            e0v[pl.ds(off, L)] = ea
            e1v[pl.ds(off, L)] = eb
            idxv[pl.ds(off, L)] = ia * KB + ib
            return c

        lax.fori_loop(0, CHUNK // L, pass1, 0)

        copies = []
        for g in range(NG):
            copies.append(pltpu.async_copy(
                zt_h.at[idxv.at[pl.ds(g * GSUB, GSUB)]],
                zv.at[pl.ds(g * GSUB, GSUB)], sem))
        for cp in copies:
            cp.wait()

        def pass2(vi, c):
            off = vi * L
            ea = e0v[pl.ds(off, L)]
            eb = e1v[pl.ds(off, L)]
            pidx = iota + off
            acc = jnp.zeros((L,), jnp.float32)
            for m in range(M):
                rz = plsc.load_gather(zv, [pidx, jnp.full((L,), m, jnp.int32)])
                cm = cv[m]
                pa = ((cm[3] * ea + cm[2]) * ea + cm[1]) * ea + cm[0]
                pb = ((cm[7] * eb + cm[6]) * eb + cm[5]) * eb + cm[4]
                t = pa * pb
                acc = acc + (t * t + cm[8]) * rz
            outv[pl.ds(off, L)] = _log_f32(acc)
            return c

        lax.fori_loop(0, CHUNK // L, pass2, 0)
        pltpu.sync_copy(outv, out_h.at[pl.ds(cbase, CHUNK)])
        return carry

    lax.fori_loop(0, NCHUNK, chunk_body, 0)


@functools.partial(
    pl.kernel,
    out_type=jax.ShapeDtypeStruct((N,), jnp.float32),
    mesh=plsc.VectorSubcoreMesh(
        core_axis_name="c", subcore_axis_name="s",
        num_cores=NC, num_subcores=NS),
    compiler_params=pltpu.CompilerParams(
        needs_layout_passes=False, use_tc_tiling_on_sc=False),
    scratch_types=[
        pltpu.VMEM((K,), jnp.float32),
        pltpu.VMEM((K,), jnp.float32),
        pltpu.VMEM((M * 9 * 128,), jnp.float32),
        pltpu.VMEM((CHUNK,), jnp.float32),
        pltpu.VMEM((CHUNK,), jnp.float32),
        pltpu.VMEM((CHUNK,), jnp.float32),
        pltpu.VMEM((CHUNK,), jnp.float32),
        pltpu.VMEM((CHUNK,), jnp.int32),
        pltpu.VMEM((CHUNK, M), jnp.float32),
        pltpu.VMEM((CHUNK,), jnp.float32),
        pltpu.SemaphoreType.DMA,
    ],
)
def _spline_sq2(x0_h, x1_h, k0_h, k1_h, par_h, zt_h, out_h, *scratch):
    _sc_kernel(x0_h, x1_h, k0_h, k1_h, par_h, zt_h, out_h, *scratch)


def kernel(x, knots, poly_params, mixture_weights, integrals_2dgrid):
    x0 = x[:, 0]
    x1 = x[:, 1]
    k0 = knots[:, 0]
    k1 = knots[:, 1]
    sw = jnp.sqrt(mixture_weights)
    c0 = poly_params[:, 0, :] * sw[:, None]
    c1 = poly_params[:, 1, :]
    epsw = (1e-30 * mixture_weights)[:, None]
    par = jnp.concatenate([c0, c1, epsw], axis=1)
    par = jnp.broadcast_to(par[:, :, None], (M, 9, 128)).reshape(-1)
    zt = (1.0 / integrals_2dgrid).transpose(1, 2, 0).reshape(KB * KB, M)
    return _spline_sq2(x0, x1, k0, k1, par, zt)


# speedup vs baseline: 163.1766x; 1.1353x over previous
import functools

import jax
import jax.numpy as jnp
from jax import lax
from jax.experimental import pallas as pl
from jax.experimental.pallas import tpu as pltpu
from jax.experimental.pallas import tpu_sc as plsc

N = 2097152
K = 1024
M = 8
KB = K - 1

NC, NS, L = 2, 16, 16
NW = NC * NS
PW = N // NW
CHUNK = 4096
NCHUNK = PW // CHUNK
NPAIR = NCHUNK // 2
GSUB = 128
NG = CHUNK // GSUB
UN1 = 4
UN2 = 2

_LN2 = 0.6931471805599453
_SQRT_HALF_HI = 1.4142135


def _log_f32(v):
    b = plsc.bitcast(v, jnp.int32)
    e = (b >> 23) - 127
    m = plsc.bitcast((b & 0x007FFFFF) | 0x3F800000, jnp.float32)
    adj = m > _SQRT_HALF_HI
    m = jnp.where(adj, m * 0.5, m)
    ef = (e + adj.astype(jnp.int32)).astype(jnp.float32)
    s = (m - 1.0) / (m + 1.0)
    s2 = s * s
    r = s * (2.0 + s2 * (2.0 / 3.0 + s2 * (2.0 / 5.0 + s2 * (2.0 / 7.0))))
    return ef * _LN2 + r


def _sc_kernel(x0_h, x1_h, k0_h, k1_h, par_h, zt_h, out_h,
               k0v, k1v, parv, xv0, xv1, e0v, e1v, idxv, zv, outv,
               semz0, semz1):
    wid = lax.axis_index("s") * NC + lax.axis_index("c")
    base0 = wid * PW

    pltpu.sync_copy(k0_h, k0v)
    pltpu.sync_copy(k1_h, k1v)
    pltpu.sync_copy(par_h, parv)

    iota = lax.iota(jnp.int32, L)
    cv = [[parv[pl.ds((m * 9 + d) * L, L)] for d in range(9)] for m in range(M)]
    mvecs = [jnp.full((L,), m, jnp.int32) for m in range(M)]
    bvecs = [jnp.full((L,), b, jnp.int32) for b in range(2)]

    def load_x(b, ci):
        cb = base0 + ci * CHUNK
        pltpu.sync_copy(x0_h.at[pl.ds(cb, CHUNK)], xv0.at[b])
        pltpu.sync_copy(x1_h.at[pl.ds(cb, CHUNK)], xv1.at[b])

    def search(kref, xa):
        base = jnp.zeros((L,), jnp.int32)
        length = K
        while length > 1:
            half = length // 2
            probe = plsc.load_gather(kref, [base + (half - 1)])
            base = jnp.where(probe < xa, base + half, base)
            length -= half
        probe = plsc.load_gather(kref, [base])
        lb = base + (probe < xa).astype(jnp.int32)
        return jnp.clip(lb - 1, 0, KB - 1)

    def pass1(b):
        def body(vi, c):
            off0 = vi * (L * UN1)
            xs = []
            for u in range(UN1):
                off = off0 + u * L
                xs.append((off, xv0[b, pl.ds(off, L)], xv1[b, pl.ds(off, L)]))
            for off, xa, xb in xs:
                ia = search(k0v, xa)
                ib = search(k1v, xb)
                e0v[b, pl.ds(off, L)] = xa - plsc.load_gather(k0v, [ia])
                e1v[b, pl.ds(off, L)] = xb - plsc.load_gather(k1v, [ib])
                idxv[b, pl.ds(off, L)] = ia * KB + ib
            return c

        lax.fori_loop(0, CHUNK // (L * UN1), body, 0)

    def issue_z(b):
        sem = semz0 if b == 0 else semz1
        for g in range(NG):
            pltpu.async_copy(
                zt_h.at[idxv.at[b, pl.ds(g * GSUB, GSUB)]],
                zv.at[b, pl.ds(g * GSUB, GSUB)], sem)

    def wait_z(b):
        sem = semz0 if b == 0 else semz1
        for g in range(NG):
            pltpu.make_async_copy(
                zt_h.at[idxv.at[b, pl.ds(g * GSUB, GSUB)]],
                zv.at[b, pl.ds(g * GSUB, GSUB)], sem).wait()

    def pass2(b, ci):
        bv = bvecs[b]

        def body(vi, c):
            off0 = vi * (L * UN2)
            for u in range(UN2):
                off = off0 + u * L
                ea = e0v[b, pl.ds(off, L)]
                eb = e1v[b, pl.ds(off, L)]
                pidx = iota + off
                acc = jnp.zeros((L,), jnp.float32)
                for m in range(M):
                    rz = plsc.load_gather(zv, [bv, pidx, mvecs[m]])
                    cm = cv[m]
                    pa = ((cm[3] * ea + cm[2]) * ea + cm[1]) * ea + cm[0]
                    pb = ((cm[7] * eb + cm[6]) * eb + cm[5]) * eb + cm[4]
                    t = pa * pb
                    acc = acc + (t * t + cm[8]) * rz
                outv[b, pl.ds(off, L)] = _log_f32(acc)
            return c

        lax.fori_loop(0, CHUNK // (L * UN2), body, 0)
        cb = base0 + ci * CHUNK
        pltpu.sync_copy(outv.at[b], out_h.at[pl.ds(cb, CHUNK)])

    def pair(hi, carry):
        ci = hi * 2
        load_x(0, ci)
        pass1(0)
        issue_z(0)

        @pl.when(hi > 0)
        def _older():
            wait_z(1)
            pass2(1, ci - 1)

        load_x(1, ci + 1)
        pass1(1)
        issue_z(1)
        wait_z(0)
        pass2(0, ci)
        return carry

    lax.fori_loop(0, NPAIR, pair, 0)
    wait_z(1)
    pass2(1, NCHUNK - 1)


@functools.partial(
    pl.kernel,
    out_type=jax.ShapeDtypeStruct((N,), jnp.float32),
    mesh=plsc.VectorSubcoreMesh(
        core_axis_name="c", subcore_axis_name="s",
        num_cores=NC, num_subcores=NS),
    compiler_params=pltpu.CompilerParams(
        needs_layout_passes=False, use_tc_tiling_on_sc=False),
    scratch_types=[
        pltpu.VMEM((K,), jnp.float32),
        pltpu.VMEM((K,), jnp.float32),
        pltpu.VMEM((M * 9 * L,), jnp.float32),
        pltpu.VMEM((2, CHUNK), jnp.float32),
        pltpu.VMEM((2, CHUNK), jnp.float32),
        pltpu.VMEM((2, CHUNK), jnp.float32),
        pltpu.VMEM((2, CHUNK), jnp.float32),
        pltpu.VMEM((2, CHUNK), jnp.int32),
        pltpu.VMEM((2, CHUNK, M), jnp.float32),
        pltpu.VMEM((2, CHUNK), jnp.float32),
        pltpu.SemaphoreType.DMA,
        pltpu.SemaphoreType.DMA,
    ],
)
def _spline_sq2(x0_h, x1_h, k0_h, k1_h, par_h, zt_h, out_h, *scratch):
    _sc_kernel(x0_h, x1_h, k0_h, k1_h, par_h, zt_h, out_h, *scratch)


def kernel(x, knots, poly_params, mixture_weights, integrals_2dgrid):
    x0 = x[:, 0]
    x1 = x[:, 1]
    k0 = knots[:, 0]
    k1 = knots[:, 1]
    sw = jnp.sqrt(mixture_weights)
    c0 = poly_params[:, 0, :] * sw[:, None]
    c1 = poly_params[:, 1, :]
    epsw = (1e-30 * mixture_weights)[:, None]
    par = jnp.concatenate([c0, c1, epsw], axis=1)
    par = jnp.broadcast_to(par[:, :, None], (M, 9, L)).reshape(-1)
    zt = (1.0 / integrals_2dgrid).transpose(1, 2, 0).reshape(KB * KB, M)
    return _spline_sq2(x0, x1, k0, k1, par, zt)


# speedup vs baseline: 194.2721x; 1.1906x over previous
import functools

import jax
import jax.numpy as jnp
from jax import lax
from jax.experimental import pallas as pl
from jax.experimental.pallas import tpu as pltpu
from jax.experimental.pallas import tpu_sc as plsc

N = 2097152
K = 1024
M = 8
KB = K - 1

NC, NS, L = 2, 16, 16
NW = NC * NS
PW = N // NW
CHUNK = 4096
NCHUNK = PW // CHUNK
NPAIR = NCHUNK // 2
GSUB = 128
NG = CHUNK // GSUB
UN1 = 4
UN2 = 4

_LN2 = 0.6931471805599453
_SQRT_HALF_HI = 1.4142135


def _log_f32(v):
    b = plsc.bitcast(v, jnp.int32)
    e = (b >> 23) - 127
    m = plsc.bitcast((b & 0x007FFFFF) | 0x3F800000, jnp.float32)
    adj = m > _SQRT_HALF_HI
    m = jnp.where(adj, m * 0.5, m)
    ef = (e + adj.astype(jnp.int32)).astype(jnp.float32)
    s = (m - 1.0) / (m + 1.0)
    s2 = s * s
    r = s * (2.0 + s2 * (2.0 / 3.0 + s2 * (2.0 / 5.0 + s2 * (2.0 / 7.0))))
    return ef * _LN2 + r


def _sc_kernel(x0_h, x1_h, k0_h, k1_h, par_h, zt_h, out_h,
               k0v, k1v, parv, xv0, xv1, e0v, e1v, idxv, zv, outv,
               semz0, semz1):
    wid = lax.axis_index("s") * NC + lax.axis_index("c")
    base0 = wid * PW

    pltpu.sync_copy(k0_h, k0v)
    pltpu.sync_copy(k1_h, k1v)
    pltpu.sync_copy(par_h, parv)

    iota = lax.iota(jnp.int32, L)
    prow = [parv[pl.ds(j * L, L)] for j in range((M * 9 + L - 1) // L)]
    scal = [prow[j // L][j % L] for j in range(M * 9)]
    cv = [[scal[m * 9 + d] for d in range(9)] for m in range(M)]
    mvecs = [jnp.full((L,), m, jnp.int32) for m in range(M)]
    bvecs = [jnp.full((L,), b, jnp.int32) for b in range(2)]

    def load_x(b, ci):
        cb = base0 + ci * CHUNK
        pltpu.sync_copy(x0_h.at[pl.ds(cb, CHUNK)], xv0.at[b])
        pltpu.sync_copy(x1_h.at[pl.ds(cb, CHUNK)], xv1.at[b])

    def search(kref, xa):
        base = jnp.zeros((L,), jnp.int32)
        length = K
        while length > 1:
            half = length // 2
            probe = plsc.load_gather(kref, [base + (half - 1)])
            base = jnp.where(probe < xa, base + half, base)
            length -= half
        probe = plsc.load_gather(kref, [base])
        lb = base + (probe < xa).astype(jnp.int32)
        return jnp.clip(lb - 1, 0, KB - 1)

    def pass1(b):
        @plsc.parallel_loop(0, CHUNK // L, unroll=UN1)
        def _p1(vi):
            off = vi * L
            xa = xv0[b, pl.ds(off, L)]
            xb = xv1[b, pl.ds(off, L)]
            ia = search(k0v, xa)
            ib = search(k1v, xb)
            e0v[b, pl.ds(off, L)] = xa - plsc.load_gather(k0v, [ia])
            e1v[b, pl.ds(off, L)] = xb - plsc.load_gather(k1v, [ib])
            idxv[b, pl.ds(off, L)] = ia * KB + ib

    def issue_z(b):
        sem = semz0 if b == 0 else semz1
        for g in range(NG):
            pltpu.async_copy(
                zt_h.at[idxv.at[b, pl.ds(g * GSUB, GSUB)]],
                zv.at[b, pl.ds(g * GSUB, GSUB)], sem)

    def wait_z(b):
        sem = semz0 if b == 0 else semz1
        for g in range(NG):
            pltpu.make_async_copy(
                zt_h.at[idxv.at[b, pl.ds(g * GSUB, GSUB)]],
                zv.at[b, pl.ds(g * GSUB, GSUB)], sem).wait()

    def pass2(b, ci):
        bv = bvecs[b]
        for m in range(M):
            cm = cv[m]

            @plsc.parallel_loop(0, CHUNK // L, unroll=UN2)
            def _pm(vi):
                off = vi * L
                ea = e0v[b, pl.ds(off, L)]
                eb = e1v[b, pl.ds(off, L)]
                rz = plsc.load_gather(zv, [bv, iota + off, mvecs[m]])
                pa = ((cm[3] * ea + cm[2]) * ea + cm[1]) * ea + cm[0]
                pb = ((cm[7] * eb + cm[6]) * eb + cm[5]) * eb + cm[4]
                t = pa * pb
                if m == 0:
                    outv[b, pl.ds(off, L)] = t * t * rz + 1e-30
                else:
                    plsc.addupdate(outv.at[b, pl.ds(off, L)], t * t * rz)

        @plsc.parallel_loop(0, CHUNK // L, unroll=UN2)
        def _lg(vi):
            off = vi * L
            outv[b, pl.ds(off, L)] = _log_f32(outv[b, pl.ds(off, L)])

        cb = base0 + ci * CHUNK
        pltpu.sync_copy(outv.at[b], out_h.at[pl.ds(cb, CHUNK)])

    def pair(hi, carry):
        ci = hi * 2
        load_x(0, ci)
        pass1(0)
        issue_z(0)

        @pl.when(hi > 0)
        def _older():
            wait_z(1)
            pass2(1, ci - 1)

        load_x(1, ci + 1)
        pass1(1)
        issue_z(1)
        wait_z(0)
        pass2(0, ci)
        return carry

    lax.fori_loop(0, NPAIR, pair, 0)
    wait_z(1)
    pass2(1, NCHUNK - 1)


@functools.partial(
    pl.kernel,
    out_type=jax.ShapeDtypeStruct((N,), jnp.float32),
    mesh=plsc.VectorSubcoreMesh(
        core_axis_name="c", subcore_axis_name="s",
        num_cores=NC, num_subcores=NS),
    compiler_params=pltpu.CompilerParams(
        needs_layout_passes=False, use_tc_tiling_on_sc=False),
    scratch_types=[
        pltpu.VMEM((K,), jnp.float32),
        pltpu.VMEM((K,), jnp.float32),
        pltpu.VMEM((128,), jnp.float32),
        pltpu.VMEM((2, CHUNK), jnp.float32),
        pltpu.VMEM((2, CHUNK), jnp.float32),
        pltpu.VMEM((2, CHUNK), jnp.float32),
        pltpu.VMEM((2, CHUNK), jnp.float32),
        pltpu.VMEM((2, CHUNK), jnp.int32),
        pltpu.VMEM((2, CHUNK, M), jnp.float32),
        pltpu.VMEM((2, CHUNK), jnp.float32),
        pltpu.SemaphoreType.DMA,
        pltpu.SemaphoreType.DMA,
    ],
)
def _spline_sq2(x0_h, x1_h, k0_h, k1_h, par_h, zt_h, out_h, *scratch):
    _sc_kernel(x0_h, x1_h, k0_h, k1_h, par_h, zt_h, out_h, *scratch)


def kernel(x, knots, poly_params, mixture_weights, integrals_2dgrid):
    x0 = x[:, 0]
    x1 = x[:, 1]
    k0 = knots[:, 0]
    k1 = knots[:, 1]
    sw = jnp.sqrt(mixture_weights)
    c0 = poly_params[:, 0, :] * sw[:, None]
    c1 = poly_params[:, 1, :]
    epsw = (1e-30 * mixture_weights)[:, None]
    par = jnp.concatenate([c0, c1, epsw], axis=1).reshape(-1)
    par = jnp.pad(par, (0, 128 - M * 9))
    zt = (1.0 / integrals_2dgrid).transpose(1, 2, 0).reshape(KB * KB, M)
    return _spline_sq2(x0, x1, k0, k1, par, zt)


# speedup vs baseline: 199.2143x; 1.0254x over previous
import functools

import jax
import jax.numpy as jnp
from jax import lax
from jax.experimental import pallas as pl
from jax.experimental.pallas import tpu as pltpu
from jax.experimental.pallas import tpu_sc as plsc

N = 2097152
K = 1024
M = 8
KB = K - 1

NC, NS, L = 2, 16, 16
NW = NC * NS
PW = N // NW
CHUNK = 4096
NCHUNK = PW // CHUNK
NPAIR = NCHUNK // 2
GSUB = 128
NG = CHUNK // GSUB
UN1 = 4
UN2 = 4

_LN2 = 0.6931471805599453
_SQRT_HALF_HI = 1.4142135


def _bf16r(v):
    b = plsc.bitcast(v, jnp.int32)
    r = b + 32767 + ((b >> 16) & 1)
    return plsc.bitcast(r & jnp.int32(-65536), jnp.float32)


def _log_f32(v):
    b = plsc.bitcast(v, jnp.int32)
    e = (b >> 23) - 127
    m = plsc.bitcast((b & 0x007FFFFF) | 0x3F800000, jnp.float32)
    adj = m > _SQRT_HALF_HI
    m = jnp.where(adj, m * 0.5, m)
    ef = (e + adj.astype(jnp.int32)).astype(jnp.float32)
    s = (m - 1.0) / (m + 1.0)
    s2 = s * s
    r = s * (2.0 + s2 * (2.0 / 3.0 + s2 * (2.0 / 5.0 + s2 * (2.0 / 7.0))))
    return ef * _LN2 + r


NCELL = KB * KB
CELLP = 1048576
TBLK = CELLP // CHUNK
TPT = TBLK // NS


def _sc_kernel(x0_h, x1_h, k0_h, k1_h, par_h, rg_h, out_h, zt_h,
               k0v, k1v, parv, xv0, xv1, e0v, e1v, idxv, zv, outv,
               semz0, semz1):
    sid = lax.axis_index("s")
    wid = sid * NC + lax.axis_index("c")
    base0 = wid * PW

    pltpu.sync_copy(k0_h, k0v)
    pltpu.sync_copy(k1_h, k1v)
    pltpu.sync_copy(par_h, parv)

    iota = lax.iota(jnp.int32, L)
    prow = [_bf16r(parv[pl.ds(j * L, L)]) for j in range((M * 9 + L - 1) // L)]
    scal = [prow[j // L][j % L] for j in range(M * 9)]
    cv = [[scal[m * 9 + d] for d in range(9)] for m in range(M)]
    mvecs = [jnp.full((L,), m, jnp.int32) for m in range(M)]
    bvecs = [jnp.full((L,), b, jnp.int32) for b in range(2)]

    tins = [xv0.at[0], xv0.at[1], xv1.at[0], xv1.at[1],
            e0v.at[0], e0v.at[1], e1v.at[0], e1v.at[1]]

    def transpose_block(i, carry):
        blk = sid * TPT + i
        c0 = blk * CHUNK
        for m in range(M):
            pltpu.async_copy(rg_h.at[m, pl.ds(c0, CHUNK)], tins[m], semz0)
        for m in range(M):
            pltpu.make_async_copy(rg_h.at[m, pl.ds(c0, CHUNK)], tins[m],
                                  semz0).wait()

        @plsc.parallel_loop(0, CHUNK // L, unroll=2)
        def _tr(j):
            cvec = iota + j * L
            for m in range(M):
                v = tins[m][pl.ds(j * L, L)]
                plsc.store_scatter(zv, [bvecs[0], cvec, mvecs[m]], v)

        pltpu.sync_copy(zv.at[0], zt_h.at[pl.ds(c0, CHUNK)])
        return carry

    lax.fori_loop(0, TPT, transpose_block, 0)
    plsc.subcore_barrier()

    def load_x(b, ci):
        cb = base0 + ci * CHUNK
        pltpu.sync_copy(x0_h.at[pl.ds(cb, CHUNK)], xv0.at[b])
        pltpu.sync_copy(x1_h.at[pl.ds(cb, CHUNK)], xv1.at[b])

    def search(kref, xa):
        base = jnp.zeros((L,), jnp.int32)
        length = K
        while length > 1:
            half = length // 2
            probe = plsc.load_gather(kref, [base + (half - 1)])
            base = jnp.where(probe < xa, base + half, base)
            length -= half
        probe = plsc.load_gather(kref, [base])
        lb = base + (probe < xa).astype(jnp.int32)
        return jnp.clip(lb - 1, 0, KB - 1)

    def pass1(b):
        @plsc.parallel_loop(0, CHUNK // L, unroll=UN1)
        def _p1(vi):
            off = vi * L
            xa = xv0[b, pl.ds(off, L)]
            xb = xv1[b, pl.ds(off, L)]
            ia = search(k0v, xa)
            ib = search(k1v, xb)
            e0v[b, pl.ds(off, L)] = xa - plsc.load_gather(k0v, [ia])
            e1v[b, pl.ds(off, L)] = xb - plsc.load_gather(k1v, [ib])
            idxv[b, pl.ds(off, L)] = ia * KB + ib

    def issue_z(b):
        sem = semz0 if b == 0 else semz1
        for g in range(NG):
            pltpu.async_copy(
                zt_h.at[idxv.at[b, pl.ds(g * GSUB, GSUB)]],
                zv.at[b, pl.ds(g * GSUB, GSUB)], sem)

    def wait_z(b):
        sem = semz0 if b == 0 else semz1
        for g in range(NG):
            pltpu.make_async_copy(
                zt_h.at[idxv.at[b, pl.ds(g * GSUB, GSUB)]],
                zv.at[b, pl.ds(g * GSUB, GSUB)], sem).wait()

    def pass2(b, ci):
        bv = bvecs[b]
        for m in range(M):
            cm = cv[m]

            @plsc.parallel_loop(0, CHUNK // L, unroll=UN2)
            def _pm(vi):
                off = vi * L
                ea = e0v[b, pl.ds(off, L)]
                eb = e1v[b, pl.ds(off, L)]
                rz = plsc.load_gather(zv, [bv, iota + off, mvecs[m]])
                ea1 = _bf16r(ea)
                ea2 = _bf16r(ea * ea)
                ea3 = _bf16r((ea * ea) * ea)
                eb1 = _bf16r(eb)
                eb2 = _bf16r(eb * eb)
                eb3 = _bf16r((eb * eb) * eb)
                pa = ((cm[0] + cm[1] * ea1) + cm[2] * ea2) + cm[3] * ea3
                pb = ((cm[4] + cm[5] * eb1) + cm[6] * eb2) + cm[7] * eb3
                t = pa * pb
                if m == 0:
                    outv[b, pl.ds(off, L)] = t * t * rz + 1e-30
                else:
                    plsc.addupdate(outv.at[b, pl.ds(off, L)], t * t * rz)

        @plsc.parallel_loop(0, CHUNK // L, unroll=UN2)
        def _lg(vi):
            off = vi * L
            outv[b, pl.ds(off, L)] = _log_f32(outv[b, pl.ds(off, L)])

        cb = base0 + ci * CHUNK
        pltpu.sync_copy(outv.at[b], out_h.at[pl.ds(cb, CHUNK)])

    def pair(hi, carry):
        ci = hi * 2
        load_x(0, ci)
        pass1(0)
        issue_z(0)

        @pl.when(hi > 0)
        def _older():
            wait_z(1)
            pass2(1, ci - 1)

        load_x(1, ci + 1)
        pass1(1)
        issue_z(1)
        wait_z(0)
        pass2(0, ci)
        return carry

    lax.fori_loop(0, NPAIR, pair, 0)
    wait_z(1)
    pass2(1, NCHUNK - 1)


@functools.partial(
    pl.kernel,
    out_type=(jax.ShapeDtypeStruct((N,), jnp.float32),
              jax.ShapeDtypeStruct((CELLP, M), jnp.float32)),
    mesh=plsc.VectorSubcoreMesh(
        core_axis_name="c", subcore_axis_name="s",
        num_cores=NC, num_subcores=NS),
    compiler_params=pltpu.CompilerParams(
        needs_layout_passes=False, use_tc_tiling_on_sc=False),
    scratch_types=[
        pltpu.VMEM((K,), jnp.float32),
        pltpu.VMEM((K,), jnp.float32),
        pltpu.VMEM((128,), jnp.float32),
        pltpu.VMEM((2, CHUNK), jnp.float32),
        pltpu.VMEM((2, CHUNK), jnp.float32),
        pltpu.VMEM((2, CHUNK), jnp.float32),
        pltpu.VMEM((2, CHUNK), jnp.float32),
        pltpu.VMEM((2, CHUNK), jnp.int32),
        pltpu.VMEM((2, CHUNK, M), jnp.float32),
        pltpu.VMEM((2, CHUNK), jnp.float32),
        pltpu.SemaphoreType.DMA,
        pltpu.SemaphoreType.DMA,
    ],
)
def _spline_sq2(x0_h, x1_h, k0_h, k1_h, par_h, rg_h, out_h, zt_h, *scratch):
    _sc_kernel(x0_h, x1_h, k0_h, k1_h, par_h, rg_h, out_h, zt_h, *scratch)


def kernel(x, knots, poly_params, mixture_weights, integrals_2dgrid):
    x0 = x[:, 0]
    x1 = x[:, 1]
    k0 = knots[:, 0]
    k1 = knots[:, 1]
    c0 = poly_params[:, 0, :]
    c1 = poly_params[:, 1, :]
    epsw = (1e-30 * mixture_weights)[:, None]
    par = jnp.concatenate([c0, c1, epsw], axis=1).reshape(-1)
    par = jnp.pad(par, (0, 128 - M * 9))
    rg = jnp.zeros((M, CELLP), jnp.float32)
    rg = rg.at[:, :NCELL].set(
        (mixture_weights[:, None, None] / integrals_2dgrid).reshape(M, NCELL))
    out, _ = _spline_sq2(x0, x1, k0, k1, par, rg)
    return out


# speedup vs baseline: 384.6399x; 1.9308x over previous
import functools

import jax
import jax.numpy as jnp
from jax import lax
from jax.experimental import pallas as pl
from jax.experimental.pallas import tpu as pltpu
from jax.experimental.pallas import tpu_sc as plsc

N = 2097152
K = 1024
M = 8
KB = K - 1

NC, NS, L = 2, 16, 16
NW = NC * NS
PW = N // NW
CHUNK = 2048
NCHUNK = PW // CHUNK
NPAIR = NCHUNK // 2
GSUB = 128
NG = CHUNK // GSUB
UN1 = 4
UN2 = 4

_LN2 = 0.6931471805599453
_SQRT_HALF_HI = 1.4142135


def _bf16r(v):
    b = plsc.bitcast(v, jnp.int32)
    r = b + 32767 + ((b >> 16) & 1)
    return plsc.bitcast(r & jnp.int32(-65536), jnp.float32)


def _log_f32(v):
    b = plsc.bitcast(v, jnp.int32)
    e = (b >> 23) - 127
    m = plsc.bitcast((b & 0x007FFFFF) | 0x3F800000, jnp.float32)
    adj = m > _SQRT_HALF_HI
    m = jnp.where(adj, m * 0.5, m)
    ef = (e + adj.astype(jnp.int32)).astype(jnp.float32)
    s = (m - 1.0) / (m + 1.0)
    s2 = s * s
    r = s * (2.0 + s2 * (2.0 / 3.0 + s2 * (2.0 / 5.0 + s2 * (2.0 / 7.0))))
    return ef * _LN2 + r


NCELL = KB * KB
CELLP = 1048576
TBLK = CELLP // CHUNK
TPT = TBLK // NS
NRW = 5


def _sc_kernel(x0_h, x1_h, k0_h, k1_h, par_h, rg_h, out_h, zt_h,
               k0v, k1v, parv, xv0, xv1, e0v, e1v, idxv, zv, outv,
               t0, t1, t2, t3, t4, t5, t6, t7, semz0, semz1):
    sid = lax.axis_index("s")
    wid = sid * NC + lax.axis_index("c")
    base0 = wid * PW

    pltpu.sync_copy(k0_h, k0v)
    pltpu.sync_copy(k1_h, k1v)
    pltpu.sync_copy(par_h, parv)

    iota = lax.iota(jnp.int32, L)
    prow = [_bf16r(parv[pl.ds(j * L, L)]) for j in range((M * 9 + L - 1) // L)]
    scal = [prow[j // L][j % L] for j in range(M * 9)]
    cv = [[scal[m * 9 + d] for d in range(9)] for m in range(M)]
    mvecs = [jnp.full((L,), m, jnp.int32) for m in range(M)]
    bvecs = [jnp.full((L,), b, jnp.int32) for b in range(2)]

    tins = [t0, t1, t2, t3, t4, t5, t6, t7]

    def transpose_block(i, carry):
        blk = sid * TPT + i
        c0 = blk * CHUNK
        rb10 = c0 >> 10
        rb = jnp.minimum(rb10, KB - NRW)
        for m in range(M):
            pltpu.async_copy(rg_h.at[m, pl.ds(rb, NRW)], tins[m], semz0)
        for m in range(M):
            pltpu.make_async_copy(rg_h.at[m, pl.ds(rb, NRW)], tins[m],
                                  semz0).wait()

        remv = (c0 - rb10 * KB) + iota
        rex = (remv >= KB).astype(jnp.int32) + (remv >= 2 * KB).astype(jnp.int32)
        row0 = (rb10 - rb) + rex
        col0 = remv - rex * KB
        cell0 = c0 + iota

        def _tr(j, rc):
            row_v, col_v, cell = rc
            rl = jnp.minimum(row_v, NRW - 1)
            cl = jnp.minimum(col_v, KB - 1)
            valid = cell < NCELL
            for m in range(M):
                v = plsc.load_gather(tins[m], [rl, cl])
                plsc.store_scatter(zv, [bvecs[0], iota + j * L, mvecs[m]], v,
                                   mask=valid)
            col_v = col_v + L
            wrap = col_v >= KB
            col_v = jnp.where(wrap, col_v - KB, col_v)
            row_v = row_v + wrap.astype(jnp.int32)
            return row_v, col_v, cell + L

        lax.fori_loop(0, CHUNK // L, _tr, (row0, col0, cell0))
        pltpu.sync_copy(zv.at[0], zt_h.at[pl.ds(c0, CHUNK)])
        return carry

    lax.fori_loop(0, TPT, transpose_block, 0)
    plsc.subcore_barrier()

    def load_x(b, ci):
        cb = base0 + ci * CHUNK
        pltpu.sync_copy(x0_h.at[pl.ds(cb, CHUNK)], xv0.at[b])
        pltpu.sync_copy(x1_h.at[pl.ds(cb, CHUNK)], xv1.at[b])

    def search(kref, xa):
        base = jnp.zeros((L,), jnp.int32)
        length = K
        while length > 1:
            half = length // 2
            probe = plsc.load_gather(kref, [base + (half - 1)])
            base = jnp.where(probe < xa, base + half, base)
            length -= half
        probe = plsc.load_gather(kref, [base])
        lb = base + (probe < xa).astype(jnp.int32)
        return jnp.clip(lb - 1, 0, KB - 1)

    def pass1(b):
        @plsc.parallel_loop(0, CHUNK // L, unroll=UN1)
        def _p1(vi):
            off = vi * L
            xa = xv0[b, pl.ds(off, L)]
            xb = xv1[b, pl.ds(off, L)]
            ia = search(k0v, xa)
            ib = search(k1v, xb)
            e0v[b, pl.ds(off, L)] = xa - plsc.load_gather(k0v, [ia])
            e1v[b, pl.ds(off, L)] = xb - plsc.load_gather(k1v, [ib])
            idxv[b, pl.ds(off, L)] = ia * KB + ib

    def issue_z(b):
        sem = semz0 if b == 0 else semz1
        for g in range(NG):
            pltpu.async_copy(
                zt_h.at[idxv.at[b, pl.ds(g * GSUB, GSUB)]],
                zv.at[b, pl.ds(g * GSUB, GSUB)], sem)

    def wait_z(b):
        sem = semz0 if b == 0 else semz1
        for g in range(NG):
            pltpu.make_async_copy(
                zt_h.at[idxv.at[b, pl.ds(g * GSUB, GSUB)]],
                zv.at[b, pl.ds(g * GSUB, GSUB)], sem).wait()

    def pass2(b, ci):
        bv = bvecs[b]
        for m in range(M):
            cm = cv[m]

            @plsc.parallel_loop(0, CHUNK // L, unroll=UN2)
            def _pm(vi):
                off = vi * L
                ea = e0v[b, pl.ds(off, L)]
                eb = e1v[b, pl.ds(off, L)]
                rz = plsc.load_gather(zv, [bv, iota + off, mvecs[m]])
                ea1 = _bf16r(ea)
                ea2 = _bf16r(ea * ea)
                ea3 = _bf16r((ea * ea) * ea)
                eb1 = _bf16r(eb)
                eb2 = _bf16r(eb * eb)
                eb3 = _bf16r((eb * eb) * eb)
                pa = ((cm[0] + cm[1] * ea1) + cm[2] * ea2) + cm[3] * ea3
                pb = ((cm[4] + cm[5] * eb1) + cm[6] * eb2) + cm[7] * eb3
                t = pa * pb
                if m == 0:
                    outv[b, pl.ds(off, L)] = t * t * rz + 1e-30
                else:
                    plsc.addupdate(outv.at[b, pl.ds(off, L)], t * t * rz)

        @plsc.parallel_loop(0, CHUNK // L, unroll=UN2)
        def _lg(vi):
            off = vi * L
            outv[b, pl.ds(off, L)] = _log_f32(outv[b, pl.ds(off, L)])

        cb = base0 + ci * CHUNK
        pltpu.sync_copy(outv.at[b], out_h.at[pl.ds(cb, CHUNK)])

    def pair(hi, carry):
        ci = hi * 2
        load_x(0, ci)
        pass1(0)
        issue_z(0)

        @pl.when(hi > 0)
        def _older():
            wait_z(1)
            pass2(1, ci - 1)

        load_x(1, ci + 1)
        pass1(1)
        issue_z(1)
        wait_z(0)
        pass2(0, ci)
        return carry

    lax.fori_loop(0, NPAIR, pair, 0)
    wait_z(1)
    pass2(1, NCHUNK - 1)


@functools.partial(
    pl.kernel,
    out_type=(jax.ShapeDtypeStruct((N,), jnp.float32),
              jax.ShapeDtypeStruct((CELLP, M), jnp.float32)),
    mesh=plsc.VectorSubcoreMesh(
        core_axis_name="c", subcore_axis_name="s",
        num_cores=NC, num_subcores=NS),
    compiler_params=pltpu.CompilerParams(
        needs_layout_passes=False, use_tc_tiling_on_sc=False),
    scratch_types=[
        pltpu.VMEM((K,), jnp.float32),
        pltpu.VMEM((K,), jnp.float32),
        pltpu.VMEM((128,), jnp.float32),
        pltpu.VMEM((2, CHUNK), jnp.float32),
        pltpu.VMEM((2, CHUNK), jnp.float32),
        pltpu.VMEM((2, CHUNK), jnp.float32),
        pltpu.VMEM((2, CHUNK), jnp.float32),
        pltpu.VMEM((2, CHUNK), jnp.int32),
        pltpu.VMEM((2, CHUNK, M), jnp.float32),
        pltpu.VMEM((2, CHUNK), jnp.float32),
        pltpu.VMEM((NRW, KB), jnp.float32),
        pltpu.VMEM((NRW, KB), jnp.float32),
        pltpu.VMEM((NRW, KB), jnp.float32),
        pltpu.VMEM((NRW, KB), jnp.float32),
        pltpu.VMEM((NRW, KB), jnp.float32),
        pltpu.VMEM((NRW, KB), jnp.float32),
        pltpu.VMEM((NRW, KB), jnp.float32),
        pltpu.VMEM((NRW, KB), jnp.float32),
        pltpu.SemaphoreType.DMA,
        pltpu.SemaphoreType.DMA,
    ],
)
def _spline_sq2(x0_h, x1_h, k0_h, k1_h, par_h, rg_h, out_h, zt_h, *scratch):
    _sc_kernel(x0_h, x1_h, k0_h, k1_h, par_h, rg_h, out_h, zt_h, *scratch)


def kernel(x, knots, poly_params, mixture_weights, integrals_2dgrid):
    x0 = x[:, 0]
    x1 = x[:, 1]
    k0 = knots[:, 0]
    k1 = knots[:, 1]
    c0 = poly_params[:, 0, :]
    c1 = poly_params[:, 1, :]
    epsw = (1e-30 * mixture_weights)[:, None]
    par = jnp.concatenate([c0, c1, epsw], axis=1).reshape(-1)
    par = jnp.pad(par, (0, 128 - M * 9))
    rg = mixture_weights[:, None, None] / integrals_2dgrid
    out, _ = _spline_sq2(x0, x1, k0, k1, par, rg)
    return out
